# baseline (speedup 1.0000x reference)
import functools

import jax
import jax.numpy as jnp
from jax import lax
from jax.experimental import pallas as pl
from jax.experimental.pallas import tpu as pltpu
from jax.experimental.pallas import tpu_sc as plsc

RAW = 40962
NEW = RAW * 4 - 6
TBL = 7 * RAW
X2N = NEW - RAW
NW = 32

RAWP4 = 10241
RAWP = 4 * RAWP4
X1P = 41472
NEWP = 164352
NEWP4 = NEWP // 4
HB4 = X2N * 2 * 16 // 128
HA4 = X1P * 32 // 128

X2_PER_W = X2N * 2 // NW
X2_CHUNK = 1920
X1_PER_W = X1P // NW
Q = NEWP // NW
QC = 1712

_mesh = plsc.VectorSubcoreMesh(
    core_axis_name="c", subcore_axis_name="s", num_cores=2, num_subcores=16)
_sc_params = pltpu.CompilerParams(
    needs_layout_passes=False, use_tc_tiling_on_sc=False)


def _worker_id():
  return lax.axis_index("s") * 2 + lax.axis_index("c")


def _blockdiag4(w):
  eye4 = jnp.eye(4, dtype=w.dtype)
  return jnp.einsum("ab,io->aibo", eye4, w).reshape(4 * w.shape[0],
                                                    4 * w.shape[1])


_RBA = 1024


def _upconv_body(x_ref, wy_ref, by_ref, wp_ref, bp_ref, y_ref, z_ref):
  xb = x_ref[...]
  for k in range(7):
    y_ref[k] = (
        jnp.dot(xb, wy_ref[k], preferred_element_type=jnp.float32)
        + by_ref[k:k + 1, :]
    )
  z_ref[...] = (
      jnp.dot(xb, wp_ref[...], preferred_element_type=jnp.float32) + bp_ref[...]
  )


def _upconv_call(x4, wy, by, wp4, bp4):
  nb = pl.cdiv(RAWP4, _RBA)
  return pl.pallas_call(
      _upconv_body,
      grid=(nb,),
      in_specs=[
          pl.BlockSpec((_RBA, 256), lambda i: (i, 0)),
          pl.BlockSpec((7, 256, 128), lambda i: (0, 0, 0)),
          pl.BlockSpec((7, 128), lambda i: (0, 0)),
          pl.BlockSpec((256, 448), lambda i: (0, 0)),
          pl.BlockSpec((1, 448), lambda i: (0, 0)),
      ],
      out_specs=[
          pl.BlockSpec((7, _RBA, 128), lambda i: (0, i, 0)),
          pl.BlockSpec((_RBA, 448), lambda i: (i, 0)),
      ],
      out_shape=[
          jax.ShapeDtypeStruct((7, RAWP4, 128), jnp.float32),
          jax.ShapeDtypeStruct((RAWP4, 448), jnp.float32),
      ],
  )(x4, wy, by, wp4, bp4)


Z16_ROWS = RAWP4 * 448 // 16
X1_LASTW = (NW - 1) * X1_PER_W
X1_VALID = RAW - X1_LASTW
X1_MS0 = 784


@functools.partial(
    pl.kernel,
    out_type=(
        jax.ShapeDtypeStruct((X1P, 32), jnp.float32),
        jax.ShapeDtypeStruct((2 * X2N, 16), jnp.float32),
    ),
    mesh=_mesh,
    scratch_types=[
        pltpu.VMEM((X2_CHUNK,), jnp.int32),
        pltpu.VMEM((X2_CHUNK, 16), jnp.float32),
        pltpu.VMEM((X1_PER_W,), jnp.int32),
        pltpu.VMEM((X1_PER_W,), jnp.int32),
        pltpu.VMEM((X1_PER_W, 32), jnp.float32),
        pltpu.SemaphoreType.DMA,
    ],
    compiler_params=_sc_params,
)
def _upgather(ytab, z16, top, down, h_a, h_b, idx2_v, buf2_v, top_v, idx1_v,
              buf1_v, sem):
  wid = _worker_id()
  def x2_chunk(c, _):
    rowbase = wid * X2_PER_W + c * X2_CHUNK
    pltpu.sync_copy(down.at[pl.ds(rowbase, X2_CHUNK)], idx2_v)
    pltpu.async_copy(z16.at[idx2_v], buf2_v, sem).wait()
    pltpu.sync_copy(buf2_v, h_b.at[pl.ds(rowbase, X2_CHUNK)])
    return 0
  lax.fori_loop(0, X2_PER_W // X2_CHUNK, x2_chunk, 0)

  tbase = wid * X1_PER_W
  lanes = lax.iota(jnp.int32, 16)

  @pl.when(wid < NW - 1)
  def _():
    pltpu.sync_copy(top.at[pl.ds(tbase, X1_PER_W)], top_v)

  @pl.when(wid == NW - 1)
  def _():
    def ms(i, _):
      top_v[pl.ds(X1_MS0 + 16 * i, 16)] = jnp.zeros((16,), jnp.int32)
      return 0
    lax.fori_loop(0, (X1_PER_W - X1_MS0) // 16, ms, 0)
    pltpu.sync_copy(top.at[pl.ds(X1_LASTW, X1_VALID)],
                    top_v.at[pl.ds(0, X1_VALID)])

  def build(j, _):
    v = plsc.load_gather(top_v, [j * 16 + lanes])
    r = ((v.astype(jnp.float32) + 0.5) * (1.0 / 7.0)).astype(jnp.int32)
    k = v - r * 7
    idx1_v[pl.ds(j * 16, 16)] = k * RAWP + r
    return 0
  lax.fori_loop(0, X1_PER_W // 16, build, 0)
  pltpu.async_copy(ytab.at[idx1_v], buf1_v, sem).wait()
  pltpu.sync_copy(buf1_v, h_a.at[pl.ds(tbase, X1_PER_W)])


_RBC = 1024


_HB_NB = HB4 // _RBC


def _proj_body(hb_ref, ha_ref, w_ref, out_ref):
  i = pl.program_id(0)
  hb = jnp.where(i < _HB_NB, hb_ref[...], ha_ref[...])
  for k in range(7):
    out_ref[k] = jnp.dot(hb, w_ref[k], preferred_element_type=jnp.float32)


def _proj_call(hb4, ha4, wb):
  nb = pl.cdiv(NEWP4, _RBC)
  return pl.pallas_call(
      _proj_body,
      grid=(nb,),
      in_specs=[
          pl.BlockSpec((_RBC, 128), lambda i: (jnp.minimum(i, _HB_NB - 1), 0)),
          pl.BlockSpec((_RBC, 128),
                       lambda i: (jnp.maximum(i - _HB_NB, 0), 0)),
          pl.BlockSpec((7, 128, 128), lambda i: (0, 0, 0)),
      ],
      out_specs=pl.BlockSpec((7, _RBC, 128), lambda i: (0, i, 0)),
      out_shape=jax.ShapeDtypeStruct((7, NEWP4, 128), jnp.float32),
  )(hb4, ha4, wb)


def _fold128(s):
  return s[:, 0:32] + s[:, 32:64] + s[:, 64:96] + s[:, 96:128]


def _normalize_packed(t, s_ref, g_ref, bt_ref):
  s = _fold128(s_ref[...])
  mean = s[0:1, :] * (1.0 / NEW)
  var = s[1:2, :] * (1.0 / NEW) - mean * mean
  a = g_ref[...] * lax.rsqrt(var + 1e-5)
  c = bt_ref[...] - mean * a
  a4 = jnp.concatenate([a, a, a, a], axis=1)
  c4 = jnp.concatenate([c, c, c, c], axis=1)
  t = t * a4 + c4
  return jnp.where(t >= 0, t, 0.2 * t)


def _bnproj_body(t_ref, s_ref, g_ref, bt_ref, w_ref, out_ref):
  hb = _normalize_packed(t_ref[...], s_ref, g_ref, bt_ref)
  for k in range(7):
    out_ref[k] = jnp.dot(hb, w_ref[k], preferred_element_type=jnp.float32)


def _bnproj_call(t4, s, g, bt, wb):
  nb = pl.cdiv(NEWP4, _RBC)
  return pl.pallas_call(
      _bnproj_body,
      grid=(nb,),
      in_specs=[
          pl.BlockSpec((_RBC, 128), lambda i: (i, 0)),
          pl.BlockSpec((2, 128), lambda i: (0, 0)),
          pl.BlockSpec((1, 32), lambda i: (0, 0)),
          pl.BlockSpec((1, 32), lambda i: (0, 0)),
          pl.BlockSpec((7, 128, 128), lambda i: (0, 0, 0)),
      ],
      out_specs=pl.BlockSpec((7, _RBC, 128), lambda i: (0, i, 0)),
      out_shape=jax.ShapeDtypeStruct((7, NEWP4, 128), jnp.float32),
  )(t4, s, g, bt, wb)


G_NCH = Q // QC
G_LASTBASE = (NW - 1) * Q + (G_NCH - 1) * QC
G_VALID7 = 7 * (NEW - G_LASTBASE)


def _make_gather7(remap):
  @functools.partial(
      pl.kernel,
      out_type=jax.ShapeDtypeStruct((NEWP, 32), jnp.float32),
      mesh=_mesh,
      scratch_types=[
          pltpu.VMEM((7 * QC,), jnp.int32),
          pltpu.VMEM((7 * QC,), jnp.int32),
          pltpu.VMEM((7, QC), jnp.int32),
          pltpu.VMEM((7, QC), jnp.int32),
          pltpu.VMEM((QC, 32), jnp.float32),
          pltpu.SemaphoreType.DMA,
          pltpu.SemaphoreType.DMA,
      ],
      name="gather7_remap" if remap else "gather7",
      compiler_params=_sc_params,
  )
  def gather7(h_tables, neigh, out, nraw0, nraw1, idxk0, idxk1, acc_v, sem_g,
              sem_w):
    wid = _worker_id()
    lanes7 = lax.iota(jnp.int32, 16) * 7
    nraws, idxks = (nraw0, nraw1), (idxk0, idxk1)

    def load_idx(t, nraw_v):
      base = wid * Q + t * QC
      if t == G_NCH - 1:
        @pl.when(wid == NW - 1)
        def _():
          def ms(i, _):
            nraw_v[pl.ds(16 * i, 16)] = jnp.zeros((16,), jnp.int32)
            return 0
          lax.fori_loop(0, 7 * QC // 16, ms, 0)
          pltpu.sync_copy(neigh.at[pl.ds(7 * G_LASTBASE, G_VALID7)],
                          nraw_v.at[pl.ds(0, G_VALID7)])

        @pl.when(wid < NW - 1)
        def _():
          pltpu.sync_copy(neigh.at[pl.ds(7 * base, 7 * QC)], nraw_v)
      else:
        pltpu.sync_copy(neigh.at[pl.ds(7 * base, 7 * QC)], nraw_v)

    def deint(nraw_v, idxk_v):
      def body(j, _):
        for k in range(7):
          v = plsc.load_gather(nraw_v, [j * 112 + k + lanes7])
          if remap:
            v = jnp.where(v < RAW, v + X2N, v - RAW)
          idxk_v[k, pl.ds(j * 16, 16)] = v + k * NEWP
        return 0
      lax.fori_loop(0, QC // 16, body, 0)

    load_idx(0, nraws[0])
    deint(nraws[0], idxks[0])
    pending_write = None
    for t in range(G_NCH):
      idxk_v = idxks[t % 2]
      base = wid * Q + t * QC
      if pending_write is not None:
        pending_write.wait()
      pltpu.async_copy(h_tables.at[idxk_v.at[0]], acc_v, sem_g).wait()
      descs = [
          pltpu.async_copy(h_tables.at[idxk_v.at[k]], acc_v, sem_g, add=True)
          for k in range(1, 7)
      ]
      if t + 1 < G_NCH:
        load_idx(t + 1, nraws[(t + 1) % 2])
        deint(nraws[(t + 1) % 2], idxks[(t + 1) % 2])
      for d in descs:
        d.wait()
      if t + 1 < G_NCH:
        pending_write = pltpu.async_copy(acc_v, out.at[pl.ds(base, QC)], sem_w)
      else:
        pltpu.sync_copy(acc_v, out.at[pl.ds(base, QC)])

  return gather7


_gather7_remap = _make_gather7(True)
_gather7_plain = _make_gather7(False)


_RBS = 2048


def _stats_accum(t_ref, acc_ref, i):
  @pl.when(i == 0)
  def _():
    acc_ref[...] = jnp.zeros_like(acc_ref)

  t = t_ref[...]
  rows = lax.broadcasted_iota(jnp.int32, t.shape, 0) + i * _RBS
  cols = lax.broadcasted_iota(jnp.int32, t.shape, 1)
  valid = rows * 4 + lax.shift_right_logical(cols, 5) < NEW
  t = jnp.where(valid, t, 0.0)
  acc_ref[0:1, :] += jnp.sum(t, axis=0, keepdims=True)
  acc_ref[1:2, :] += jnp.sum(t * t, axis=0, keepdims=True)


def _stats_body(t_ref, o_ref, acc_ref):
  i = pl.program_id(0)
  _stats_accum(t_ref, acc_ref, i)

  @pl.when(i == pl.num_programs(0) - 1)
  def _():
    o_ref[...] = acc_ref[...]


def _stats_call(t4):
  nb = pl.cdiv(NEWP4, _RBS)
  return pl.pallas_call(
      _stats_body,
      grid=(nb,),
      in_specs=[pl.BlockSpec((_RBS, 128), lambda i: (i, 0))],
      out_specs=pl.BlockSpec((2, 128), lambda i: (0, 0)),
      out_shape=jax.ShapeDtypeStruct((2, 128), jnp.float32),
      scratch_shapes=[pltpu.VMEM((2, 128), jnp.float32)],
  )(t4)


def _stats_ac_body(t_ref, g_ref, bt_ref, o_ref, acc_ref):
  i = pl.program_id(0)
  _stats_accum(t_ref, acc_ref, i)

  @pl.when(i == pl.num_programs(0) - 1)
  def _():
    s = _fold128(acc_ref[...])
    mean = s[0:1, :] * (1.0 / NEW)
    var = s[1:2, :] * (1.0 / NEW) - mean * mean
    a = g_ref[...] * lax.rsqrt(var + 1e-5)
    c = bt_ref[...] - mean * a
    o_ref[...] = jnp.concatenate([a, c], axis=0)


def _stats_ac_call(t4, g, bt):
  nb = pl.cdiv(NEWP4, _RBS)
  return pl.pallas_call(
      _stats_ac_body,
      grid=(nb,),
      in_specs=[
          pl.BlockSpec((_RBS, 128), lambda i: (i, 0)),
          pl.BlockSpec((1, 32), lambda i: (0, 0)),
          pl.BlockSpec((1, 32), lambda i: (0, 0)),
      ],
      out_specs=pl.BlockSpec((2, 32), lambda i: (0, 0)),
      out_shape=jax.ShapeDtypeStruct((2, 32), jnp.float32),
      scratch_shapes=[pltpu.VMEM((2, 128), jnp.float32)],
  )(t4, g, bt)


G_FVALID = NEW - G_LASTBASE


@functools.partial(
    pl.kernel,
    out_type=jax.ShapeDtypeStruct((NEW, 32), jnp.float32),
    mesh=_mesh,
    scratch_types=[
        pltpu.VMEM((2, 32), jnp.float32),
        pltpu.VMEM((QC, 32), jnp.float32),
        pltpu.SemaphoreType.DMA,
    ],
    name="finalize",
    compiler_params=_sc_params,
)
def _finalize(t_hbm, ac_hbm, out_hbm, ac_v, buf_v, sem):
  wid = _worker_id()
  pltpu.sync_copy(ac_hbm, ac_v)
  a_lo = ac_v[0, pl.ds(0, 16)]
  a_hi = ac_v[0, pl.ds(16, 16)]
  c_lo = ac_v[1, pl.ds(0, 16)]
  c_hi = ac_v[1, pl.ds(16, 16)]

  def chunk(t, _):
    base = wid * Q + t * QC
    pltpu.sync_copy(t_hbm.at[pl.ds(base, QC)], buf_v)

    def rows(j, _):
      for rr in range(4):
        r = j * 4 + rr
        u = buf_v[r, pl.ds(0, 16)] * a_lo + c_lo
        buf_v[r, pl.ds(0, 16)] = jnp.maximum(u, 0.2 * u)
        u = buf_v[r, pl.ds(16, 16)] * a_hi + c_hi
        buf_v[r, pl.ds(16, 16)] = jnp.maximum(u, 0.2 * u)
      return 0
    lax.fori_loop(0, QC // 4, rows, 0)

    @pl.when(base + QC <= NEW)
    def _():
      pltpu.sync_copy(buf_v, out_hbm.at[pl.ds(base, QC)])

    @pl.when(base + QC > NEW)
    def _():
      pltpu.sync_copy(buf_v.at[pl.ds(0, G_FVALID)],
                      out_hbm.at[pl.ds(G_LASTBASE, G_FVALID)])
    return 0
  lax.fori_loop(0, Q // QC, chunk, 0)


def kernel(x, neigh_orders, upconv_top_index, upconv_down_index, W_up, b_up,
           W1, b1, g1, beta1, W2, b2, g2, beta2):
  del b1, b2
  f32 = jnp.float32
  w_pair = 0.5 * (W_up[:, 0::2] + W_up[:, 1::2])
  b_pair = 0.5 * (b_up[0::2] + b_up[1::2])
  eye4 = jnp.eye(4, dtype=f32)
  wyk = W_up.reshape(64, 7, 32).transpose(1, 0, 2)
  wy = jnp.einsum("ab,kio->kaibo", eye4, wyk).reshape(7, 256, 128)
  by = jnp.tile(b_up.reshape(7, 32), (1, 4))
  wp4 = _blockdiag4(w_pair)
  bp4 = jnp.tile(b_pair, 4).reshape(1, 448)
  w1r = W1.reshape(7, 32, 32)
  wb1 = jnp.einsum("ab,kio->kaibo", eye4, w1r).reshape(7, 128, 128)
  w2r = W2.reshape(7, 32, 32)
  wb2 = jnp.einsum("ab,kio->kaibo", eye4, w2r).reshape(7, 128, 128)

  x4 = jnp.concatenate([x, jnp.zeros((2, 64), f32)]).reshape(RAWP4, 256)

  ytabp, z4 = _upconv_call(x4, wy, by, wp4, bp4)
  ytab = ytabp.reshape(7 * RAWP, 32)
  z16 = z4.reshape(Z16_ROWS, 16)

  h_a, h_b = _upgather(ytab, z16, upconv_top_index, upconv_down_index)

  ht1 = _proj_call(h_b.reshape(HB4, 128), h_a.reshape(HA4, 128),
                   wb1).reshape(7 * NEWP, 32)
  out1 = _gather7_remap(ht1, neigh_orders)
  out1p = out1.reshape(NEWP4, 128)
  s1 = _stats_call(out1p)

  ht2 = _bnproj_call(out1p, s1, g1.reshape(1, 32), beta1.reshape(1, 32),
                     wb2).reshape(7 * NEWP, 32)
  out2 = _gather7_plain(ht2, neigh_orders)
  ac2 = _stats_ac_call(out2.reshape(NEWP4, 128), g2.reshape(1, 32),
                       beta2.reshape(1, 32))
  return _finalize(out2, ac2)


# speedup vs baseline: 1.5956x; 1.5956x over previous
import functools

import jax
import jax.numpy as jnp
from jax import lax
from jax.experimental import pallas as pl
from jax.experimental.pallas import tpu as pltpu
from jax.experimental.pallas import tpu_sc as plsc

RAW = 40962
NEW = RAW * 4 - 6
TBL = 7 * RAW
X2N = NEW - RAW
NW = 32

RAWP4 = 10248
RAWP = 4 * RAWP4
X1P = 41472
NEWP = 164352
NEWP4 = NEWP // 4
HB4 = X2N * 2 * 16 // 128
HA4 = X1P * 32 // 128

X2_PER_W = X2N * 2 // NW
X2_CHUNK = 1920
X1_PER_W = X1P // NW
Q = NEWP // NW
QC = 1712

_mesh = plsc.VectorSubcoreMesh(
    core_axis_name="c", subcore_axis_name="s", num_cores=2, num_subcores=16)
_sc_params = pltpu.CompilerParams(
    needs_layout_passes=False, use_tc_tiling_on_sc=False)


def _worker_id():
  return lax.axis_index("s") * 2 + lax.axis_index("c")


def _blockdiag4(w):
  eye4 = jnp.eye(4, dtype=w.dtype)
  return jnp.einsum("ab,io->aibo", eye4, w).reshape(4 * w.shape[0],
                                                    4 * w.shape[1])


_RBA = 1024


def _upconv_body(x_ref, wy_ref, by_ref, wp_ref, bp_ref, y_ref, z_ref):
  xb = x_ref[...]
  for k in range(7):
    y_ref[k] = (
        jnp.dot(xb, wy_ref[k], preferred_element_type=jnp.float32)
        + by_ref[k:k + 1, :]
    )
  z_ref[...] = (
      jnp.dot(xb, wp_ref[...], preferred_element_type=jnp.float32) + bp_ref[...]
  )


def _upconv_call(x4, wy, by, wp4, bp4):
  nb = pl.cdiv(RAWP4, _RBA)
  return pl.pallas_call(
      _upconv_body,
      grid=(nb,),
      in_specs=[
          pl.BlockSpec((_RBA, 256), lambda i: (i, 0)),
          pl.BlockSpec((7, 256, 128), lambda i: (0, 0, 0)),
          pl.BlockSpec((7, 128), lambda i: (0, 0)),
          pl.BlockSpec((256, 448), lambda i: (0, 0)),
          pl.BlockSpec((1, 448), lambda i: (0, 0)),
      ],
      out_specs=[
          pl.BlockSpec((7, _RBA, 128), lambda i: (0, i, 0)),
          pl.BlockSpec((_RBA, 448), lambda i: (i, 0)),
      ],
      out_shape=[
          jax.ShapeDtypeStruct((7, RAWP4, 128), jnp.float32),
          jax.ShapeDtypeStruct((RAWP4, 448), jnp.float32),
      ],
  )(x4, wy, by, wp4, bp4)


Z16_ROWS = RAWP4 * 448 // 16
X1_LASTW = (NW - 1) * X1_PER_W
X1_VALID = RAW - X1_LASTW
X1_MS0 = 784


@functools.partial(
    pl.kernel,
    out_type=(
        jax.ShapeDtypeStruct((X1P, 32), jnp.float32),
        jax.ShapeDtypeStruct((2 * X2N, 16), jnp.float32),
    ),
    mesh=_mesh,
    scratch_types=[
        pltpu.VMEM((X2_CHUNK,), jnp.int32),
        pltpu.VMEM((X2_CHUNK, 16), jnp.float32),
        pltpu.VMEM((X1_PER_W,), jnp.int32),
        pltpu.VMEM((X1_PER_W,), jnp.int32),
        pltpu.VMEM((X1_PER_W, 32), jnp.float32),
        pltpu.SemaphoreType.DMA,
    ],
    compiler_params=_sc_params,
)
def _upgather(ytab, z16, top, down, h_a, h_b, idx2_v, buf2_v, top_v, idx1_v,
              buf1_v, sem):
  wid = _worker_id()
  def x2_chunk(c, _):
    rowbase = wid * X2_PER_W + c * X2_CHUNK
    pltpu.sync_copy(down.at[pl.ds(rowbase, X2_CHUNK)], idx2_v)
    pltpu.async_copy(z16.at[idx2_v], buf2_v, sem).wait()
    pltpu.sync_copy(buf2_v, h_b.at[pl.ds(rowbase, X2_CHUNK)])
    return 0
  lax.fori_loop(0, X2_PER_W // X2_CHUNK, x2_chunk, 0)

  tbase = wid * X1_PER_W
  lanes = lax.iota(jnp.int32, 16)

  @pl.when(wid < NW - 1)
  def _():
    pltpu.sync_copy(top.at[pl.ds(tbase, X1_PER_W)], top_v)

  @pl.when(wid == NW - 1)
  def _():
    def ms(i, _):
      top_v[pl.ds(X1_MS0 + 16 * i, 16)] = jnp.zeros((16,), jnp.int32)
      return 0
    lax.fori_loop(0, (X1_PER_W - X1_MS0) // 16, ms, 0)
    pltpu.sync_copy(top.at[pl.ds(X1_LASTW, X1_VALID)],
                    top_v.at[pl.ds(0, X1_VALID)])

  def build(j, _):
    v = plsc.load_gather(top_v, [j * 16 + lanes])
    r = ((v.astype(jnp.float32) + 0.5) * (1.0 / 7.0)).astype(jnp.int32)
    k = v - r * 7
    idx1_v[pl.ds(j * 16, 16)] = k * RAWP + r
    return 0
  lax.fori_loop(0, X1_PER_W // 16, build, 0)
  pltpu.async_copy(ytab.at[idx1_v], buf1_v, sem).wait()
  pltpu.sync_copy(buf1_v, h_a.at[pl.ds(tbase, X1_PER_W)])


_RBC = 1024


_HB_NB = HB4 // _RBC


def _proj_body(hb_ref, ha_ref, w_ref, out_ref):
  i = pl.program_id(0)
  hb = jnp.where(i < _HB_NB, hb_ref[...], ha_ref[...])
  for k in range(7):
    out_ref[k] = jnp.dot(hb, w_ref[k], preferred_element_type=jnp.float32)


def _proj_call(hb4, ha4, wb):
  nb = pl.cdiv(NEWP4, _RBC)
  return pl.pallas_call(
      _proj_body,
      grid=(nb,),
      in_specs=[
          pl.BlockSpec((_RBC, 128), lambda i: (jnp.minimum(i, _HB_NB - 1), 0)),
          pl.BlockSpec((_RBC, 128),
                       lambda i: (jnp.maximum(i - _HB_NB, 0), 0)),
          pl.BlockSpec((7, 128, 128), lambda i: (0, 0, 0)),
      ],
      out_specs=pl.BlockSpec((7, _RBC, 128), lambda i: (0, i, 0)),
      out_shape=jax.ShapeDtypeStruct((7, NEWP4, 128), jnp.float32),
  )(hb4, ha4, wb)


def _fold128(s):
  return s[:, 0:32] + s[:, 32:64] + s[:, 64:96] + s[:, 96:128]


def _normalize_packed(t, s_ref, g_ref, bt_ref):
  s = _fold128(s_ref[...])
  mean = s[0:1, :] * (1.0 / NEW)
  var = s[1:2, :] * (1.0 / NEW) - mean * mean
  a = g_ref[...] * lax.rsqrt(var + 1e-5)
  c = bt_ref[...] - mean * a
  a4 = jnp.concatenate([a, a, a, a], axis=1)
  c4 = jnp.concatenate([c, c, c, c], axis=1)
  t = t * a4 + c4
  return jnp.where(t >= 0, t, 0.2 * t)


def _bnproj_body(t_ref, s_ref, g_ref, bt_ref, w_ref, out_ref):
  hb = _normalize_packed(t_ref[...], s_ref, g_ref, bt_ref)
  for k in range(7):
    out_ref[k] = jnp.dot(hb, w_ref[k], preferred_element_type=jnp.float32)


def _bnproj_call(t4, s, g, bt, wb):
  nb = pl.cdiv(NEWP4, _RBC)
  return pl.pallas_call(
      _bnproj_body,
      grid=(nb,),
      in_specs=[
          pl.BlockSpec((_RBC, 128), lambda i: (i, 0)),
          pl.BlockSpec((2, 128), lambda i: (0, 0)),
          pl.BlockSpec((1, 32), lambda i: (0, 0)),
          pl.BlockSpec((1, 32), lambda i: (0, 0)),
          pl.BlockSpec((7, 128, 128), lambda i: (0, 0, 0)),
      ],
      out_specs=pl.BlockSpec((7, _RBC, 128), lambda i: (0, i, 0)),
      out_shape=jax.ShapeDtypeStruct((7, NEWP4, 128), jnp.float32),
  )(t4, s, g, bt, wb)


G_NCH = Q // QC
G_LASTBASE = (NW - 1) * Q + (G_NCH - 1) * QC
G_VALID7 = 7 * (NEW - G_LASTBASE)


def _make_gather7(remap):
  @functools.partial(
      pl.kernel,
      out_type=jax.ShapeDtypeStruct((NEWP, 32), jnp.float32),
      mesh=_mesh,
      scratch_types=[
          pltpu.VMEM((7 * QC,), jnp.int32),
          pltpu.VMEM((7 * QC,), jnp.int32),
          pltpu.VMEM((7, QC), jnp.int32),
          pltpu.VMEM((7, QC), jnp.int32),
          pltpu.VMEM((QC, 32), jnp.float32),
          pltpu.SemaphoreType.DMA,
          pltpu.SemaphoreType.DMA,
      ],
      name="gather7_remap" if remap else "gather7",
      compiler_params=_sc_params,
  )
  def gather7(h_tables, neigh, out, nraw0, nraw1, idxk0, idxk1, acc_v, sem_g,
              sem_w):
    wid = _worker_id()
    lanes7 = lax.iota(jnp.int32, 16) * 7
    nraws, idxks = (nraw0, nraw1), (idxk0, idxk1)

    def load_idx(t, nraw_v):
      base = wid * Q + t * QC
      if t == G_NCH - 1:
        @pl.when(wid == NW - 1)
        def _():
          def ms(i, _):
            nraw_v[pl.ds(16 * i, 16)] = jnp.zeros((16,), jnp.int32)
            return 0
          lax.fori_loop(0, 7 * QC // 16, ms, 0)
          pltpu.sync_copy(neigh.at[pl.ds(7 * G_LASTBASE, G_VALID7)],
                          nraw_v.at[pl.ds(0, G_VALID7)])

        @pl.when(wid < NW - 1)
        def _():
          pltpu.sync_copy(neigh.at[pl.ds(7 * base, 7 * QC)], nraw_v)
      else:
        pltpu.sync_copy(neigh.at[pl.ds(7 * base, 7 * QC)], nraw_v)

    def deint(nraw_v, idxk_v):
      def body(j, _):
        for k in range(7):
          v = plsc.load_gather(nraw_v, [j * 112 + k + lanes7])
          if remap:
            v = jnp.where(v < RAW, v + X2N, v - RAW)
          idxk_v[k, pl.ds(j * 16, 16)] = v + k * NEWP
        return 0
      lax.fori_loop(0, QC // 16, body, 0)

    load_idx(0, nraws[0])
    deint(nraws[0], idxks[0])
    pending_write = None
    for t in range(G_NCH):
      idxk_v = idxks[t % 2]
      base = wid * Q + t * QC
      if pending_write is not None:
        pending_write.wait()
      pltpu.async_copy(h_tables.at[idxk_v.at[0]], acc_v, sem_g).wait()
      descs = [
          pltpu.async_copy(h_tables.at[idxk_v.at[k]], acc_v, sem_g, add=True)
          for k in range(1, 7)
      ]
      if t + 1 < G_NCH:
        load_idx(t + 1, nraws[(t + 1) % 2])
        deint(nraws[(t + 1) % 2], idxks[(t + 1) % 2])
      for d in descs:
        d.wait()
      if t + 1 < G_NCH:
        pending_write = pltpu.async_copy(acc_v, out.at[pl.ds(base, QC)], sem_w)
      else:
        pltpu.sync_copy(acc_v, out.at[pl.ds(base, QC)])

  return gather7


_gather7_remap = _make_gather7(True)
_gather7_plain = _make_gather7(False)


_RBS = 2048


def _stats_accum(t_ref, acc_ref, i):
  @pl.when(i == 0)
  def _():
    acc_ref[...] = jnp.zeros_like(acc_ref)

  t = t_ref[...]
  rows = lax.broadcasted_iota(jnp.int32, t.shape, 0) + i * _RBS
  cols = lax.broadcasted_iota(jnp.int32, t.shape, 1)
  valid = rows * 4 + lax.shift_right_logical(cols, 5) < NEW
  t = jnp.where(valid, t, 0.0)
  acc_ref[0:1, :] += jnp.sum(t, axis=0, keepdims=True)
  acc_ref[1:2, :] += jnp.sum(t * t, axis=0, keepdims=True)


def _stats_body(t_ref, o_ref, acc_ref):
  i = pl.program_id(0)
  _stats_accum(t_ref, acc_ref, i)

  @pl.when(i == pl.num_programs(0) - 1)
  def _():
    o_ref[...] = acc_ref[...]


def _stats_call(t4):
  nb = pl.cdiv(NEWP4, _RBS)
  return pl.pallas_call(
      _stats_body,
      grid=(nb,),
      in_specs=[pl.BlockSpec((_RBS, 128), lambda i: (i, 0))],
      out_specs=pl.BlockSpec((2, 128), lambda i: (0, 0)),
      out_shape=jax.ShapeDtypeStruct((2, 128), jnp.float32),
      scratch_shapes=[pltpu.VMEM((2, 128), jnp.float32)],
  )(t4)


def _stats_ac_body(t_ref, g_ref, bt_ref, o_ref, acc_ref):
  i = pl.program_id(0)
  _stats_accum(t_ref, acc_ref, i)

  @pl.when(i == pl.num_programs(0) - 1)
  def _():
    s = _fold128(acc_ref[...])
    mean = s[0:1, :] * (1.0 / NEW)
    var = s[1:2, :] * (1.0 / NEW) - mean * mean
    a = g_ref[...] * lax.rsqrt(var + 1e-5)
    c = bt_ref[...] - mean * a
    o_ref[...] = jnp.concatenate([a, c], axis=0)


def _stats_ac_call(t4, g, bt):
  nb = pl.cdiv(NEWP4, _RBS)
  return pl.pallas_call(
      _stats_ac_body,
      grid=(nb,),
      in_specs=[
          pl.BlockSpec((_RBS, 128), lambda i: (i, 0)),
          pl.BlockSpec((1, 32), lambda i: (0, 0)),
          pl.BlockSpec((1, 32), lambda i: (0, 0)),
      ],
      out_specs=pl.BlockSpec((2, 32), lambda i: (0, 0)),
      out_shape=jax.ShapeDtypeStruct((2, 32), jnp.float32),
      scratch_shapes=[pltpu.VMEM((2, 128), jnp.float32)],
  )(t4, g, bt)


G_FVALID = NEW - G_LASTBASE


@functools.partial(
    pl.kernel,
    out_type=jax.ShapeDtypeStruct((NEW, 32), jnp.float32),
    mesh=_mesh,
    scratch_types=[
        pltpu.VMEM((2, 32), jnp.float32),
        pltpu.VMEM((QC, 32), jnp.float32),
        pltpu.SemaphoreType.DMA,
    ],
    name="finalize",
    compiler_params=_sc_params,
)
def _finalize(t_hbm, ac_hbm, out_hbm, ac_v, buf_v, sem):
  wid = _worker_id()
  pltpu.sync_copy(ac_hbm, ac_v)
  a_lo = ac_v[0, pl.ds(0, 16)]
  a_hi = ac_v[0, pl.ds(16, 16)]
  c_lo = ac_v[1, pl.ds(0, 16)]
  c_hi = ac_v[1, pl.ds(16, 16)]

  def chunk(t, _):
    base = wid * Q + t * QC
    pltpu.sync_copy(t_hbm.at[pl.ds(base, QC)], buf_v)

    def rows(j, _):
      for rr in range(4):
        r = j * 4 + rr
        u = buf_v[r, pl.ds(0, 16)] * a_lo + c_lo
        buf_v[r, pl.ds(0, 16)] = jnp.maximum(u, 0.2 * u)
        u = buf_v[r, pl.ds(16, 16)] * a_hi + c_hi
        buf_v[r, pl.ds(16, 16)] = jnp.maximum(u, 0.2 * u)
      return 0
    lax.fori_loop(0, QC // 4, rows, 0)

    @pl.when(base + QC <= NEW)
    def _():
      pltpu.sync_copy(buf_v, out_hbm.at[pl.ds(base, QC)])

    @pl.when(base + QC > NEW)
    def _():
      pltpu.sync_copy(buf_v.at[pl.ds(0, G_FVALID)],
                      out_hbm.at[pl.ds(G_LASTBASE, G_FVALID)])
    return 0
  lax.fori_loop(0, Q // QC, chunk, 0)


def kernel(x, neigh_orders, upconv_top_index, upconv_down_index, W_up, b_up,
           W1, b1, g1, beta1, W2, b2, g2, beta2):
  del b1, b2
  f32 = jnp.float32
  w_pair = 0.5 * (W_up[:, 0::2] + W_up[:, 1::2])
  b_pair = 0.5 * (b_up[0::2] + b_up[1::2])
  eye4 = jnp.eye(4, dtype=f32)
  wyk = W_up.reshape(64, 7, 32).transpose(1, 0, 2)
  wy = jnp.einsum("ab,kio->kaibo", eye4, wyk).reshape(7, 256, 128)
  by = jnp.tile(b_up.reshape(7, 32), (1, 4))
  wp4 = _blockdiag4(w_pair)
  bp4 = jnp.tile(b_pair, 4).reshape(1, 448)
  w1r = W1.reshape(7, 32, 32)
  wb1 = jnp.einsum("ab,kio->kaibo", eye4, w1r).reshape(7, 128, 128)
  w2r = W2.reshape(7, 32, 32)
  wb2 = jnp.einsum("ab,kio->kaibo", eye4, w2r).reshape(7, 128, 128)

  x4 = jnp.concatenate(
      [x, jnp.zeros((RAWP - RAW, 64), f32)]).reshape(RAWP4, 256)

  ytabp, z4 = _upconv_call(x4, wy, by, wp4, bp4)
  ytab = ytabp.reshape(7 * RAWP, 32)
  z16 = z4.reshape(Z16_ROWS, 16)

  h_a, h_b = _upgather(ytab, z16, upconv_top_index, upconv_down_index)

  ht1 = _proj_call(h_b.reshape(HB4, 128), h_a.reshape(HA4, 128),
                   wb1).reshape(7 * NEWP, 32)
  out1 = _gather7_remap(ht1, neigh_orders)
  out1p = out1.reshape(NEWP4, 128)
  s1 = _stats_call(out1p)

  ht2 = _bnproj_call(out1p, s1, g1.reshape(1, 32), beta1.reshape(1, 32),
                     wb2).reshape(7 * NEWP, 32)
  out2 = _gather7_plain(ht2, neigh_orders)
  ac2 = _stats_ac_call(out2.reshape(NEWP4, 128), g2.reshape(1, 32),
                       beta2.reshape(1, 32))
  return _finalize(out2, ac2)


# speedup vs baseline: 1.6489x; 1.0334x over previous
import functools

import jax
import jax.numpy as jnp
from jax import lax
from jax.experimental import pallas as pl
from jax.experimental.pallas import tpu as pltpu
from jax.experimental.pallas import tpu_sc as plsc

RAW = 40962
NEW = RAW * 4 - 6
TBL = 7 * RAW
X2N = NEW - RAW
NW = 32

RAWP4 = 10248
RAWP = 4 * RAWP4
X1P = 41472
NEWP = 164352
NEWP4 = NEWP // 4
HB4 = X2N * 2 * 16 // 128
HA4 = X1P * 32 // 128

X2_PER_W = X2N * 2 // NW
X2_CHUNK = 1920
X1_PER_W = X1P // NW
Q = NEWP // NW
QC = 1712

_mesh = plsc.VectorSubcoreMesh(
    core_axis_name="c", subcore_axis_name="s", num_cores=2, num_subcores=16)
_sc_params = pltpu.CompilerParams(
    needs_layout_passes=False, use_tc_tiling_on_sc=False)


def _worker_id():
  return lax.axis_index("s") * 2 + lax.axis_index("c")


def _blockdiag4(w):
  eye4 = jnp.eye(4, dtype=w.dtype)
  return jnp.einsum("ab,io->aibo", eye4, w).reshape(4 * w.shape[0],
                                                    4 * w.shape[1])


_RBA = 1024


def _upconv_body(x_ref, wy_ref, by_ref, wp_ref, bp_ref, y_ref, z_ref):
  xb = x_ref[...]
  for k in range(7):
    y_ref[k] = (
        jnp.dot(xb, wy_ref[k], preferred_element_type=jnp.float32)
        + by_ref[k:k + 1, :]
    )
  z_ref[...] = (
      jnp.dot(xb, wp_ref[...], preferred_element_type=jnp.float32) + bp_ref[...]
  )


def _upconv_call(x4, wy, by, wp4, bp4):
  nb = pl.cdiv(RAWP4, _RBA)
  return pl.pallas_call(
      _upconv_body,
      grid=(nb,),
      in_specs=[
          pl.BlockSpec((_RBA, 256), lambda i: (i, 0)),
          pl.BlockSpec((7, 256, 128), lambda i: (0, 0, 0)),
          pl.BlockSpec((7, 128), lambda i: (0, 0)),
          pl.BlockSpec((256, 448), lambda i: (0, 0)),
          pl.BlockSpec((1, 448), lambda i: (0, 0)),
      ],
      out_specs=[
          pl.BlockSpec((7, _RBA, 128), lambda i: (0, i, 0)),
          pl.BlockSpec((_RBA, 448), lambda i: (i, 0)),
      ],
      out_shape=[
          jax.ShapeDtypeStruct((7, RAWP4, 128), jnp.float32),
          jax.ShapeDtypeStruct((RAWP4, 448), jnp.float32),
      ],
  )(x4, wy, by, wp4, bp4)


Z16_ROWS = RAWP4 * 448 // 16
X1_LASTW = (NW - 1) * X1_PER_W
X1_VALID = RAW - X1_LASTW
X1_MS0 = 784


@functools.partial(
    pl.kernel,
    out_type=(
        jax.ShapeDtypeStruct((X1P, 32), jnp.float32),
        jax.ShapeDtypeStruct((2 * X2N, 16), jnp.float32),
    ),
    mesh=_mesh,
    scratch_types=[
        pltpu.VMEM((X2_CHUNK,), jnp.int32),
        pltpu.VMEM((X2_CHUNK, 16), jnp.float32),
        pltpu.VMEM((X1_PER_W,), jnp.int32),
        pltpu.VMEM((X1_PER_W,), jnp.int32),
        pltpu.VMEM((X1_PER_W, 32), jnp.float32),
        pltpu.SemaphoreType.DMA,
    ],
    compiler_params=_sc_params,
)
def _upgather(ytab, z16, top, down, h_a, h_b, idx2_v, buf2_v, top_v, idx1_v,
              buf1_v, sem):
  wid = _worker_id()
  def x2_chunk(c, _):
    rowbase = wid * X2_PER_W + c * X2_CHUNK
    pltpu.sync_copy(down.at[pl.ds(rowbase, X2_CHUNK)], idx2_v)
    pltpu.async_copy(z16.at[idx2_v], buf2_v, sem).wait()
    pltpu.sync_copy(buf2_v, h_b.at[pl.ds(rowbase, X2_CHUNK)])
    return 0
  lax.fori_loop(0, X2_PER_W // X2_CHUNK, x2_chunk, 0)

  tbase = wid * X1_PER_W
  lanes = lax.iota(jnp.int32, 16)

  @pl.when(wid < NW - 1)
  def _():
    pltpu.sync_copy(top.at[pl.ds(tbase, X1_PER_W)], top_v)

  @pl.when(wid == NW - 1)
  def _():
    def ms(i, _):
      top_v[pl.ds(X1_MS0 + 16 * i, 16)] = jnp.zeros((16,), jnp.int32)
      return 0
    lax.fori_loop(0, (X1_PER_W - X1_MS0) // 16, ms, 0)
    pltpu.sync_copy(top.at[pl.ds(X1_LASTW, X1_VALID)],
                    top_v.at[pl.ds(0, X1_VALID)])

  def build(j, _):
    v = plsc.load_gather(top_v, [j * 16 + lanes])
    r = ((v.astype(jnp.float32) + 0.5) * (1.0 / 7.0)).astype(jnp.int32)
    k = v - r * 7
    idx1_v[pl.ds(j * 16, 16)] = k * RAWP + r
    return 0
  lax.fori_loop(0, X1_PER_W // 16, build, 0)
  pltpu.async_copy(ytab.at[idx1_v], buf1_v, sem).wait()
  pltpu.sync_copy(buf1_v, h_a.at[pl.ds(tbase, X1_PER_W)])


_RBC = 2048


_HB_NB = HB4 // _RBC


def _proj_body(hb_ref, ha_ref, w_ref, out_ref):
  i = pl.program_id(0)
  hb = jnp.where(i < _HB_NB, hb_ref[...], ha_ref[...])
  for k in range(7):
    out_ref[k] = jnp.dot(hb, w_ref[k], preferred_element_type=jnp.float32)


def _proj_call(hb4, ha4, wb):
  nb = pl.cdiv(NEWP4, _RBC)
  return pl.pallas_call(
      _proj_body,
      grid=(nb,),
      in_specs=[
          pl.BlockSpec((_RBC, 128), lambda i: (jnp.minimum(i, _HB_NB - 1), 0)),
          pl.BlockSpec((_RBC, 128),
                       lambda i: (jnp.maximum(i - _HB_NB, 0), 0)),
          pl.BlockSpec((7, 128, 128), lambda i: (0, 0, 0)),
      ],
      out_specs=pl.BlockSpec((7, _RBC, 128), lambda i: (0, i, 0)),
      out_shape=jax.ShapeDtypeStruct((7, NEWP4, 128), jnp.float32),
  )(hb4, ha4, wb)


def _fold128(s):
  return s[:, 0:32] + s[:, 32:64] + s[:, 64:96] + s[:, 96:128]


def _normalize_packed(t, s_ref, g_ref, bt_ref):
  s = _fold128(s_ref[...])
  mean = s[0:1, :] * (1.0 / NEW)
  var = s[1:2, :] * (1.0 / NEW) - mean * mean
  a = g_ref[...] * lax.rsqrt(var + 1e-5)
  c = bt_ref[...] - mean * a
  a4 = jnp.concatenate([a, a, a, a], axis=1)
  c4 = jnp.concatenate([c, c, c, c], axis=1)
  t = t * a4 + c4
  return jnp.where(t >= 0, t, 0.2 * t)


def _bnproj_body(t_ref, s_ref, g_ref, bt_ref, w_ref, out_ref):
  hb = _normalize_packed(t_ref[...], s_ref, g_ref, bt_ref)
  for k in range(7):
    out_ref[k] = jnp.dot(hb, w_ref[k], preferred_element_type=jnp.float32)


def _bnproj_call(t4, s, g, bt, wb):
  nb = pl.cdiv(NEWP4, _RBC)
  return pl.pallas_call(
      _bnproj_body,
      grid=(nb,),
      in_specs=[
          pl.BlockSpec((_RBC, 128), lambda i: (i, 0)),
          pl.BlockSpec((2, 128), lambda i: (0, 0)),
          pl.BlockSpec((1, 32), lambda i: (0, 0)),
          pl.BlockSpec((1, 32), lambda i: (0, 0)),
          pl.BlockSpec((7, 128, 128), lambda i: (0, 0, 0)),
      ],
      out_specs=pl.BlockSpec((7, _RBC, 128), lambda i: (0, i, 0)),
      out_shape=jax.ShapeDtypeStruct((7, NEWP4, 128), jnp.float32),
  )(t4, s, g, bt, wb)


G_NCH = Q // QC
G_LASTBASE = (NW - 1) * Q + (G_NCH - 1) * QC
G_VALID7 = 7 * (NEW - G_LASTBASE)


def _make_gather7(remap):
  @functools.partial(
      pl.kernel,
      out_type=jax.ShapeDtypeStruct((NEWP, 32), jnp.float32),
      mesh=_mesh,
      scratch_types=[
          pltpu.VMEM((7 * QC,), jnp.int32),
          pltpu.VMEM((7 * QC,), jnp.int32),
          pltpu.VMEM((7, QC), jnp.int32),
          pltpu.VMEM((7, QC), jnp.int32),
          pltpu.VMEM((QC, 32), jnp.float32),
          pltpu.SemaphoreType.DMA,
          pltpu.SemaphoreType.DMA,
      ],
      name="gather7_remap" if remap else "gather7",
      compiler_params=_sc_params,
  )
  def gather7(h_tables, neigh, out, nraw0, nraw1, idxk0, idxk1, acc_v, sem_g,
              sem_w):
    wid = _worker_id()
    lanes7 = lax.iota(jnp.int32, 16) * 7
    nraws, idxks = (nraw0, nraw1), (idxk0, idxk1)

    def load_idx(t, nraw_v):
      base = wid * Q + t * QC
      if t == G_NCH - 1:
        @pl.when(wid == NW - 1)
        def _():
          def ms(i, _):
            nraw_v[pl.ds(16 * i, 16)] = jnp.zeros((16,), jnp.int32)
            return 0
          lax.fori_loop(0, 7 * QC // 16, ms, 0)
          pltpu.sync_copy(neigh.at[pl.ds(7 * G_LASTBASE, G_VALID7)],
                          nraw_v.at[pl.ds(0, G_VALID7)])

        @pl.when(wid < NW - 1)
        def _():
          pltpu.sync_copy(neigh.at[pl.ds(7 * base, 7 * QC)], nraw_v)
      else:
        pltpu.sync_copy(neigh.at[pl.ds(7 * base, 7 * QC)], nraw_v)

    def deint(nraw_v, idxk_v):
      def body(j, _):
        for k in range(7):
          v = plsc.load_gather(nraw_v, [j * 112 + k + lanes7])
          if remap:
            v = jnp.where(v < RAW, v + X2N, v - RAW)
          idxk_v[k, pl.ds(j * 16, 16)] = v + k * NEWP
        return 0
      lax.fori_loop(0, QC // 16, body, 0)

    load_idx(0, nraws[0])
    deint(nraws[0], idxks[0])
    pending_write = None
    for t in range(G_NCH):
      idxk_v = idxks[t % 2]
      base = wid * Q + t * QC
      if pending_write is not None:
        pending_write.wait()
      pltpu.async_copy(h_tables.at[idxk_v.at[0]], acc_v, sem_g).wait()
      descs = [
          pltpu.async_copy(h_tables.at[idxk_v.at[k]], acc_v, sem_g, add=True)
          for k in range(1, 7)
      ]
      if t + 1 < G_NCH:
        load_idx(t + 1, nraws[(t + 1) % 2])
        deint(nraws[(t + 1) % 2], idxks[(t + 1) % 2])
      for d in descs:
        d.wait()
      if t + 1 < G_NCH:
        pending_write = pltpu.async_copy(acc_v, out.at[pl.ds(base, QC)], sem_w)
      else:
        pltpu.sync_copy(acc_v, out.at[pl.ds(base, QC)])

  return gather7


_gather7_remap = _make_gather7(True)
_gather7_plain = _make_gather7(False)


_RBS = 2048


def _stats_accum(t_ref, acc_ref, i):
  @pl.when(i == 0)
  def _():
    acc_ref[...] = jnp.zeros_like(acc_ref)

  t = t_ref[...]
  rows = lax.broadcasted_iota(jnp.int32, t.shape, 0) + i * _RBS
  cols = lax.broadcasted_iota(jnp.int32, t.shape, 1)
  valid = rows * 4 + lax.shift_right_logical(cols, 5) < NEW
  t = jnp.where(valid, t, 0.0)
  acc_ref[0:1, :] += jnp.sum(t, axis=0, keepdims=True)
  acc_ref[1:2, :] += jnp.sum(t * t, axis=0, keepdims=True)


def _stats_body(t_ref, o_ref, acc_ref):
  i = pl.program_id(0)
  _stats_accum(t_ref, acc_ref, i)

  @pl.when(i == pl.num_programs(0) - 1)
  def _():
    o_ref[...] = acc_ref[...]


def _stats_call(t4):
  nb = pl.cdiv(NEWP4, _RBS)
  return pl.pallas_call(
      _stats_body,
      grid=(nb,),
      in_specs=[pl.BlockSpec((_RBS, 128), lambda i: (i, 0))],
      out_specs=pl.BlockSpec((2, 128), lambda i: (0, 0)),
      out_shape=jax.ShapeDtypeStruct((2, 128), jnp.float32),
      scratch_shapes=[pltpu.VMEM((2, 128), jnp.float32)],
  )(t4)


def _stats_ac_body(t_ref, g_ref, bt_ref, o_ref, acc_ref):
  i = pl.program_id(0)
  _stats_accum(t_ref, acc_ref, i)

  @pl.when(i == pl.num_programs(0) - 1)
  def _():
    s = _fold128(acc_ref[...])
    mean = s[0:1, :] * (1.0 / NEW)
    var = s[1:2, :] * (1.0 / NEW) - mean * mean
    a = g_ref[...] * lax.rsqrt(var + 1e-5)
    c = bt_ref[...] - mean * a
    o_ref[...] = jnp.concatenate([a, c], axis=0)


def _stats_ac_call(t4, g, bt):
  nb = pl.cdiv(NEWP4, _RBS)
  return pl.pallas_call(
      _stats_ac_body,
      grid=(nb,),
      in_specs=[
          pl.BlockSpec((_RBS, 128), lambda i: (i, 0)),
          pl.BlockSpec((1, 32), lambda i: (0, 0)),
          pl.BlockSpec((1, 32), lambda i: (0, 0)),
      ],
      out_specs=pl.BlockSpec((2, 32), lambda i: (0, 0)),
      out_shape=jax.ShapeDtypeStruct((2, 32), jnp.float32),
      scratch_shapes=[pltpu.VMEM((2, 128), jnp.float32)],
  )(t4, g, bt)


G_FVALID = NEW - G_LASTBASE


@functools.partial(
    pl.kernel,
    out_type=jax.ShapeDtypeStruct((NEW, 32), jnp.float32),
    mesh=_mesh,
    scratch_types=[
        pltpu.VMEM((2, 32), jnp.float32),
        pltpu.VMEM((QC, 32), jnp.float32),
        pltpu.SemaphoreType.DMA,
    ],
    name="finalize",
    compiler_params=_sc_params,
)
def _finalize(t_hbm, ac_hbm, out_hbm, ac_v, buf_v, sem):
  wid = _worker_id()
  pltpu.sync_copy(ac_hbm, ac_v)
  a_lo = ac_v[0, pl.ds(0, 16)]
  a_hi = ac_v[0, pl.ds(16, 16)]
  c_lo = ac_v[1, pl.ds(0, 16)]
  c_hi = ac_v[1, pl.ds(16, 16)]

  def chunk(t, _):
    base = wid * Q + t * QC
    pltpu.sync_copy(t_hbm.at[pl.ds(base, QC)], buf_v)

    def rows(j, _):
      for rr in range(4):
        r = j * 4 + rr
        u = buf_v[r, pl.ds(0, 16)] * a_lo + c_lo
        buf_v[r, pl.ds(0, 16)] = jnp.maximum(u, 0.2 * u)
        u = buf_v[r, pl.ds(16, 16)] * a_hi + c_hi
        buf_v[r, pl.ds(16, 16)] = jnp.maximum(u, 0.2 * u)
      return 0
    lax.fori_loop(0, QC // 4, rows, 0)

    @pl.when(base + QC <= NEW)
    def _():
      pltpu.sync_copy(buf_v, out_hbm.at[pl.ds(base, QC)])

    @pl.when(base + QC > NEW)
    def _():
      pltpu.sync_copy(buf_v.at[pl.ds(0, G_FVALID)],
                      out_hbm.at[pl.ds(G_LASTBASE, G_FVALID)])
    return 0
  lax.fori_loop(0, Q // QC, chunk, 0)


def kernel(x, neigh_orders, upconv_top_index, upconv_down_index, W_up, b_up,
           W1, b1, g1, beta1, W2, b2, g2, beta2):
  del b1, b2
  f32 = jnp.float32
  w_pair = 0.5 * (W_up[:, 0::2] + W_up[:, 1::2])
  b_pair = 0.5 * (b_up[0::2] + b_up[1::2])
  eye4 = jnp.eye(4, dtype=f32)
  wyk = W_up.reshape(64, 7, 32).transpose(1, 0, 2)
  wy = jnp.einsum("ab,kio->kaibo", eye4, wyk).reshape(7, 256, 128)
  by = jnp.tile(b_up.reshape(7, 32), (1, 4))
  wp4 = _blockdiag4(w_pair)
  bp4 = jnp.tile(b_pair, 4).reshape(1, 448)
  w1r = W1.reshape(7, 32, 32)
  wb1 = jnp.einsum("ab,kio->kaibo", eye4, w1r).reshape(7, 128, 128)
  w2r = W2.reshape(7, 32, 32)
  wb2 = jnp.einsum("ab,kio->kaibo", eye4, w2r).reshape(7, 128, 128)

  x4 = jnp.concatenate(
      [x, jnp.zeros((RAWP - RAW, 64), f32)]).reshape(RAWP4, 256)

  ytabp, z4 = _upconv_call(x4, wy, by, wp4, bp4)
  ytab = ytabp.reshape(7 * RAWP, 32)
  z16 = z4.reshape(Z16_ROWS, 16)

  h_a, h_b = _upgather(ytab, z16, upconv_top_index, upconv_down_index)

  ht1 = _proj_call(h_b.reshape(HB4, 128), h_a.reshape(HA4, 128),
                   wb1).reshape(7 * NEWP, 32)
  out1 = _gather7_remap(ht1, neigh_orders)
  out1p = out1.reshape(NEWP4, 128)
  s1 = _stats_call(out1p)

  ht2 = _bnproj_call(out1p, s1, g1.reshape(1, 32), beta1.reshape(1, 32),
                     wb2).reshape(7 * NEWP, 32)
  out2 = _gather7_plain(ht2, neigh_orders)
  ac2 = _stats_ac_call(out2.reshape(NEWP4, 128), g2.reshape(1, 32),
                       beta2.reshape(1, 32))
  return _finalize(out2, ac2)
